# baseline (speedup 1.0000x reference)
import jax
import jax.numpy as jnp
from jax import lax
from jax.experimental import pallas as pl
from jax.experimental.pallas import tpu as pltpu
from jax.experimental.pallas import tpu_sc as plsc

B = 16384
F = 64
NUM_CORES = 2
NUM_SUBCORES = 16
NW = NUM_CORES * NUM_SUBCORES
BPW = B // NW
GROUPS = BPW // 16


def _body(user_hbm, item_hbm, uf_hbm, if_hbm, ub_hbm, ib_hbm, out_hbm,
          idx_u, idx_i, uf_v, if_v, ub_v, ib_v, out_v, tr_v,
          sem_u, sem_i, sem_ub, sem_ib):
    wid = lax.axis_index("s") * NUM_CORES + lax.axis_index("c")
    base = wid * BPW

    pltpu.sync_copy(user_hbm.at[pl.ds(base, BPW)], idx_u)
    cu = pltpu.async_copy(uf_hbm.at[idx_u], uf_v, sem_u)
    cub = pltpu.async_copy(ub_hbm.at[idx_u], ub_v, sem_ub)
    pltpu.sync_copy(item_hbm.at[pl.ds(base, BPW)], idx_i)
    ci = pltpu.async_copy(if_hbm.at[idx_i], if_v, sem_i)
    cib = pltpu.async_copy(ib_hbm.at[idx_i], ib_v, sem_ib)
    cu.wait()
    ci.wait()
    cub.wait()
    cib.wait()

    col16 = lax.iota(jnp.int32, 16) * 16

    def group(g, carry):
        row0 = g * 16
        for e in range(16):
            r = row0 + e
            acc = uf_v[r, pl.ds(0, 16)] * if_v[r, pl.ds(0, 16)]
            for k in range(1, 4):
                acc = acc + uf_v[r, pl.ds(k * 16, 16)] * if_v[r, pl.ds(k * 16, 16)]
            plsc.store_scatter(tr_v, [col16 + e], acc)
        tot = tr_v[pl.ds(0, 16)]
        for j in range(1, 16):
            tot = tot + tr_v[pl.ds(j * 16, 16)]
        tot = tot + ub_v[pl.ds(row0, 16)] + ib_v[pl.ds(row0, 16)]
        out_v[pl.ds(row0, 16)] = tot
        return carry

    lax.fori_loop(0, GROUPS, group, 0)
    pltpu.sync_copy(out_v, out_hbm.at[pl.ds(base, BPW)])


def kernel(user, item, user_factors, item_factors, user_bias, item_bias):
    mesh = plsc.VectorSubcoreMesh(core_axis_name="c", subcore_axis_name="s")
    k = pl.kernel(
        _body,
        out_type=jax.ShapeDtypeStruct((B,), jnp.float32),
        mesh=mesh,
        compiler_params=pltpu.CompilerParams(
            needs_layout_passes=False, use_tc_tiling_on_sc=False),
        scratch_types=[
            pltpu.VMEM((BPW,), jnp.int32),
            pltpu.VMEM((BPW,), jnp.int32),
            pltpu.VMEM((BPW, F), jnp.float32),
            pltpu.VMEM((BPW, F), jnp.float32),
            pltpu.VMEM((BPW,), jnp.float32),
            pltpu.VMEM((BPW,), jnp.float32),
            pltpu.VMEM((BPW,), jnp.float32),
            pltpu.VMEM((256,), jnp.float32),
            pltpu.SemaphoreType.DMA,
            pltpu.SemaphoreType.DMA,
            pltpu.SemaphoreType.DMA,
            pltpu.SemaphoreType.DMA,
        ],
    )
    return k(user.astype(jnp.int32), item.astype(jnp.int32),
             user_factors, item_factors,
             user_bias.reshape(-1), item_bias.reshape(-1))
